# baseline (speedup 1.0000x reference)
import functools

import jax
import jax.numpy as jnp
from jax import lax
from jax.experimental import pallas as pl
from jax.experimental.pallas import tpu as pltpu
from jax.experimental.pallas import tpu_sc as plsc

F32 = jnp.float32


def _sc_geometry():
    try:
        info = plsc.get_sparse_core_info()
        return info.num_cores, info.num_subcores
    except Exception:
        return 2, 16


def _build_sc_kernels(n, e):
    nc, ns = _sc_geometry()
    nw = nc * ns
    K = 128
    CP = (-(-e // (nw * K)) + 3) // 4 * 4
    erows = nw * CP
    HCP = CP // 2
    assert CP % 2 == 0 and HCP % 2 == 0
    ZC = 2048
    npad = -(-(n + 1) // ZC) * ZC
    nzseg = npad // ZC
    assert nzseg <= ns
    mesh = plsc.VectorSubcoreMesh(core_axis_name="c", subcore_axis_name="s")

    @functools.partial(
        pl.kernel,
        out_type=jax.ShapeDtypeStruct((nc * npad,), F32),
        mesh=mesh,
        scratch_types=[
            pltpu.VMEM_SHARED((npad,), F32),
            pltpu.VMEM((K,), jnp.int32),
            pltpu.VMEM((K,), F32),
            pltpu.VMEM((ZC,), F32),
        ],
    )
    def deg_kernel(dstm_hbm, zeros_hbm, ones_hbm, out_hbm, acc, idxv, onesv,
                   stage):
        c = lax.axis_index("c")
        s = lax.axis_index("s")
        wid = s * nc + c

        @pl.when(s < nzseg)
        def _zero():
            pltpu.sync_copy(zeros_hbm, stage)
            pltpu.sync_copy(stage, acc.at[pl.ds(s * ZC, ZC)])

        pltpu.sync_copy(ones_hbm, onesv)
        plsc.subcore_barrier()

        def body(i, carry):
            pltpu.sync_copy(dstm_hbm.at[wid * CP + i], idxv)
            pltpu.sync_copy(onesv, acc.at[idxv], add=True)
            return carry

        lax.fori_loop(0, CP, body, 0)
        plsc.subcore_barrier()

        @pl.when(s < nzseg)
        def _readout():
            pltpu.sync_copy(acc.at[pl.ds(s * ZC, ZC)], stage)
            pltpu.sync_copy(stage, out_hbm.at[pl.ds(c * npad + s * ZC, ZC)])

    RC = 80
    nrc = n // RC
    assert n % RC == 0 and RC <= K
    rfull, rrem = divmod(nrc, ns)

    @functools.partial(
        pl.kernel,
        out_type=jax.ShapeDtypeStruct((nc * n, 128), F32),
        mesh=mesh,
        scratch_types=[
            pltpu.VMEM_SHARED((n + 8, 128), F32),
            pltpu.VMEM((HCP, K), jnp.int32),
            pltpu.VMEM((HCP, K), jnp.int32),
            pltpu.VMEM((K, 128), F32),
            pltpu.VMEM((K, 128), F32),
            pltpu.SemaphoreType.DMA,
            pltpu.SemaphoreType.DMA,
        ],
    )
    def agg_kernel(y_hbm, srcm_hbm, dstm_hbm, zrows_hbm, out_hbm,
                   acc, srcall, dstall, rows0, rows1, sem0, sem1):
        c = lax.axis_index("c")
        s = lax.axis_index("s")
        wid = s * nc + c

        nrchunk = rfull + jnp.where(s < rrem, 1, 0)
        pltpu.sync_copy(zrows_hbm, rows0.at[pl.ds(0, RC)])

        def zero_chunk(i, carry):
            r0 = (i * ns + s) * RC
            pltpu.sync_copy(rows0.at[pl.ds(0, RC)], acc.at[pl.ds(r0, RC)])
            return carry

        lax.fori_loop(0, nrchunk, zero_chunk, 0)

        @pl.when(s == 0)
        def _zero_dummy():
            pltpu.sync_copy(rows0.at[pl.ds(0, 8)], acc.at[pl.ds(n, 8)])

        plsc.subcore_barrier()

        for h in range(2):
            base = wid * CP + h * HCP
            pltpu.sync_copy(srcm_hbm.at[pl.ds(base, HCP)], srcall)
            pltpu.sync_copy(dstm_hbm.at[pl.ds(base, HCP)], dstall)
            def body(i, carry):
                pltpu.sync_copy(y_hbm.at[srcall.at[i]], rows0)
                pltpu.sync_copy(rows0, acc.at[dstall.at[i]], add=True)
                return carry

            lax.fori_loop(0, HCP, body, 0)

        plsc.subcore_barrier()

        def read_chunk(i, carry):
            r0 = (i * ns + s) * RC
            pltpu.sync_copy(acc.at[pl.ds(r0, RC)], rows0.at[pl.ds(0, RC)])
            pltpu.sync_copy(rows0.at[pl.ds(0, RC)],
                            out_hbm.at[pl.ds(c * n + r0, RC)])
            return carry

        lax.fori_loop(0, nrchunk, read_chunk, 0)

    return deg_kernel, agg_kernel, nc, npad, K, RC, erows


def _tc1_body(x_ref, w1_ref, degt_ref, y1_ref, dinv_ref):
    d = degt_ref[:, 0:1] + degt_ref[:, 1:2] + 1.0
    dv = lax.rsqrt(d)
    m1 = jnp.dot(x_ref[...], w1_ref[...], preferred_element_type=F32)
    y1_ref[...] = dv * m1
    dinv_ref[...] = dv


def _tc2_body(a1_ref, y1_ref, dinv_ref, w2_ref, b1_ref, y2_ref):
    dv = dinv_ref[...]
    z1 = dv * (a1_ref[0] + a1_ref[1] + y1_ref[...]) + b1_ref[...]
    h1 = jnp.maximum(z1, 0.0)
    m2 = jnp.dot(h1, w2_ref[...], preferred_element_type=F32)
    y2_ref[...] = dv * m2


def _make_tc3_body(num_blocks, num_graphs):
    def body(a2_ref, y2_ref, dinv_ref, b2_ref, batch_ref,
             wc1_ref, bc1_ref, wc2_ref, bc2_ref, out_ref, psum, cnt):
        i = pl.program_id(0)

        @pl.when(i == 0)
        def _init():
            psum[...] = jnp.zeros_like(psum)
            cnt[...] = jnp.zeros_like(cnt)

        dv = dinv_ref[...]
        h2 = jnp.maximum(
            dv * (a2_ref[0] + a2_ref[1] + y2_ref[...]) + b2_ref[...], 0.0)
        gids = lax.broadcasted_iota(jnp.int32, (1, num_graphs), 1)
        onehot = (batch_ref[...] == gids).astype(F32)
        dn = (((0,), (0,)), ((), ()))
        psum[...] += lax.dot_general(onehot, h2, dn,
                                     preferred_element_type=F32)
        cnt[...] += lax.dot_general(onehot, jnp.ones_like(h2), dn,
                                    preferred_element_type=F32)

        @pl.when(i == num_blocks - 1)
        def _final():
            p = psum[...] / jnp.maximum(cnt[...], 1.0)
            z = jnp.maximum(
                jnp.dot(p, wc1_ref[...], preferred_element_type=F32)
                + bc1_ref[...], 0.0)
            out_ref[...] = (jnp.dot(z, wc2_ref[...],
                                    preferred_element_type=F32)
                            + bc2_ref[...])

    return body


def kernel(x, edge_index, batch, W1, b1, W2, b2, Wc1, bc1, Wc2, bc2):
    n, dmod = x.shape
    e = edge_index.shape[1]
    num_graphs = 64
    out_dim = Wc2.shape[1]
    h = W1.shape[1]
    assert dmod == 128 and h == 128

    deg_kernel, agg_kernel, nc, npad, K, RC, erows = _build_sc_kernels(n, e)

    src = edge_index[0]
    dst = edge_index[1]
    epad = erows * K
    srcm = jnp.concatenate(
        [src, jnp.zeros((epad - e,), jnp.int32)]).reshape(erows, K)
    dstm = jnp.concatenate(
        [dst, jnp.full((epad - e,), n, jnp.int32)]).reshape(erows, K)
    zeros_seg = jnp.zeros((2048,), F32)
    ones_k = jnp.ones((K,), F32)
    zrows = jnp.zeros((RC, 128), F32)

    degp = deg_kernel(dstm, zeros_seg, ones_k)
    degt = jnp.transpose(degp.reshape(nc, npad)[:, :n])

    RB = 2000
    nb = n // RB
    grid = (nb,)

    y1, dinv = pl.pallas_call(
        _tc1_body,
        grid=grid,
        in_specs=[
            pl.BlockSpec((RB, 128), lambda i: (i, 0)),
            pl.BlockSpec((128, 128), lambda i: (0, 0)),
            pl.BlockSpec((RB, nc), lambda i: (i, 0)),
        ],
        out_specs=[
            pl.BlockSpec((RB, 128), lambda i: (i, 0)),
            pl.BlockSpec((RB, 1), lambda i: (i, 0)),
        ],
        out_shape=[
            jax.ShapeDtypeStruct((n, 128), F32),
            jax.ShapeDtypeStruct((n, 1), F32),
        ],
    )(x, W1, degt)

    acc1 = agg_kernel(y1, srcm, dstm, zrows).reshape(nc, n, 128)

    y2 = pl.pallas_call(
        _tc2_body,
        grid=grid,
        in_specs=[
            pl.BlockSpec((nc, RB, 128), lambda i: (0, i, 0)),
            pl.BlockSpec((RB, 128), lambda i: (i, 0)),
            pl.BlockSpec((RB, 1), lambda i: (i, 0)),
            pl.BlockSpec((128, 128), lambda i: (0, 0)),
            pl.BlockSpec((1, 128), lambda i: (0, 0)),
        ],
        out_specs=pl.BlockSpec((RB, 128), lambda i: (i, 0)),
        out_shape=jax.ShapeDtypeStruct((n, 128), F32),
    )(acc1, y1, dinv, W2, b1.reshape(1, -1))

    acc2 = agg_kernel(y2, srcm, dstm, zrows).reshape(nc, n, 128)

    out = pl.pallas_call(
        _make_tc3_body(nb, num_graphs),
        grid=grid,
        in_specs=[
            pl.BlockSpec((nc, RB, 128), lambda i: (0, i, 0)),
            pl.BlockSpec((RB, 128), lambda i: (i, 0)),
            pl.BlockSpec((RB, 1), lambda i: (i, 0)),
            pl.BlockSpec((1, 128), lambda i: (0, 0)),
            pl.BlockSpec((RB, 1), lambda i: (i, 0)),
            pl.BlockSpec((128, Wc1.shape[1]), lambda i: (0, 0)),
            pl.BlockSpec((1, Wc1.shape[1]), lambda i: (0, 0)),
            pl.BlockSpec((Wc2.shape[0], out_dim), lambda i: (0, 0)),
            pl.BlockSpec((1, out_dim), lambda i: (0, 0)),
        ],
        out_specs=pl.BlockSpec((num_graphs, out_dim), lambda i: (0, 0)),
        out_shape=jax.ShapeDtypeStruct((num_graphs, out_dim), F32),
        scratch_shapes=[
            pltpu.VMEM((num_graphs, 128), F32),
            pltpu.VMEM((num_graphs, 128), F32),
        ],
    )(acc2, y2, dinv, b2.reshape(1, -1), batch.reshape(-1, 1),
      Wc1, bc1.reshape(1, -1), Wc2, bc2.reshape(1, -1))

    return out


# speedup vs baseline: 1.1897x; 1.1897x over previous
import functools

import jax
import jax.numpy as jnp
from jax import lax
from jax.experimental import pallas as pl
from jax.experimental.pallas import tpu as pltpu
from jax.experimental.pallas import tpu_sc as plsc

F32 = jnp.float32


def _sc_geometry():
    try:
        info = plsc.get_sparse_core_info()
        return info.num_cores, info.num_subcores
    except Exception:
        return 2, 16


def _build_sc_kernels(n, e):
    nc, ns = _sc_geometry()
    nw = nc * ns
    K = 128
    CP = (-(-e // (nw * K)) + 3) // 4 * 4
    erows = nw * CP
    HCP = CP // 2
    assert CP % 2 == 0 and HCP % 2 == 0
    ZC = 2048
    npad = -(-(n + 1) // ZC) * ZC
    nzseg = npad // ZC
    assert nzseg <= ns
    mesh = plsc.VectorSubcoreMesh(core_axis_name="c", subcore_axis_name="s")

    @functools.partial(
        pl.kernel,
        out_type=jax.ShapeDtypeStruct((nc * npad,), F32),
        mesh=mesh,
        scratch_types=[
            pltpu.VMEM_SHARED((npad,), F32),
            pltpu.VMEM((K,), jnp.int32),
            pltpu.VMEM((K,), F32),
            pltpu.VMEM((ZC,), F32),
        ],
    )
    def deg_kernel(dstm_hbm, zeros_hbm, ones_hbm, out_hbm, acc, idxv, onesv,
                   stage):
        c = lax.axis_index("c")
        s = lax.axis_index("s")
        wid = s * nc + c

        @pl.when(s < nzseg)
        def _zero():
            pltpu.sync_copy(zeros_hbm, stage)
            pltpu.sync_copy(stage, acc.at[pl.ds(s * ZC, ZC)])

        pltpu.sync_copy(ones_hbm, onesv)
        plsc.subcore_barrier()

        def body(i, carry):
            pltpu.sync_copy(dstm_hbm.at[wid * CP + i], idxv)
            pltpu.sync_copy(onesv, acc.at[idxv], add=True)
            return carry

        lax.fori_loop(0, CP, body, 0)
        plsc.subcore_barrier()

        @pl.when(s < nzseg)
        def _readout():
            pltpu.sync_copy(acc.at[pl.ds(s * ZC, ZC)], stage)
            pltpu.sync_copy(stage, out_hbm.at[pl.ds(c * npad + s * ZC, ZC)])

    RC = 80
    nrc = n // RC
    assert n % RC == 0 and RC <= K
    rfull, rrem = divmod(nrc, ns)

    KB = 256
    CPB = (CP * K) // KB
    assert (CP * K) % KB == 0

    @functools.partial(
        pl.kernel,
        out_type=jax.ShapeDtypeStruct((nc * n, 128), F32),
        mesh=mesh,
        scratch_types=[
            pltpu.VMEM_SHARED((n + 8, 128), F32),
            pltpu.VMEM((KB,), jnp.int32),
            pltpu.VMEM((KB,), jnp.int32),
            pltpu.VMEM((KB, 128), F32),
        ],
    )
    def agg_kernel(y_hbm, src_hbm, dst_hbm, zrows_hbm, out_hbm,
                   acc, idxs, idxd, rows):
        c = lax.axis_index("c")
        s = lax.axis_index("s")
        wid = s * nc + c

        nrchunk = rfull + jnp.where(s < rrem, 1, 0)
        pltpu.sync_copy(zrows_hbm, rows.at[pl.ds(0, RC)])

        def zero_chunk(i, carry):
            r0 = (i * ns + s) * RC
            pltpu.sync_copy(rows.at[pl.ds(0, RC)], acc.at[pl.ds(r0, RC)])
            return carry

        lax.fori_loop(0, nrchunk, zero_chunk, 0)

        @pl.when(s == 0)
        def _zero_dummy():
            pltpu.sync_copy(rows.at[pl.ds(0, 8)], acc.at[pl.ds(n, 8)])

        plsc.subcore_barrier()

        def body(i, carry):
            base = (i * nw + wid) * KB
            pltpu.sync_copy(src_hbm.at[pl.ds(base, KB)], idxs)
            pltpu.sync_copy(dst_hbm.at[pl.ds(base, KB)], idxd)
            pltpu.sync_copy(y_hbm.at[idxs], rows)
            pltpu.sync_copy(rows, acc.at[idxd], add=True)
            return carry

        lax.fori_loop(0, CPB, body, 0)
        plsc.subcore_barrier()

        def read_chunk(i, carry):
            r0 = (i * ns + s) * RC
            pltpu.sync_copy(acc.at[pl.ds(r0, RC)], rows.at[pl.ds(0, RC)])
            pltpu.sync_copy(rows.at[pl.ds(0, RC)],
                            out_hbm.at[pl.ds(c * n + r0, RC)])
            return carry

        lax.fori_loop(0, nrchunk, read_chunk, 0)

    return deg_kernel, agg_kernel, nc, npad, K, RC, erows


def _tc1_body(x_ref, w1_ref, degt_ref, y1_ref, dinv_ref):
    d = degt_ref[:, 0:1] + degt_ref[:, 1:2] + 1.0
    dv = lax.rsqrt(d)
    m1 = jnp.dot(x_ref[...], w1_ref[...], preferred_element_type=F32)
    y1_ref[...] = dv * m1
    dinv_ref[...] = dv


def _tc2_body(a1_ref, y1_ref, dinv_ref, w2_ref, b1_ref, y2_ref):
    dv = dinv_ref[...]
    z1 = dv * (a1_ref[0] + a1_ref[1] + y1_ref[...]) + b1_ref[...]
    h1 = jnp.maximum(z1, 0.0)
    m2 = jnp.dot(h1, w2_ref[...], preferred_element_type=F32)
    y2_ref[...] = dv * m2


def _make_tc3_body(num_blocks, num_graphs):
    def body(a2_ref, y2_ref, dinv_ref, b2_ref, batch_ref,
             wc1_ref, bc1_ref, wc2_ref, bc2_ref, out_ref, psum, cnt):
        i = pl.program_id(0)

        @pl.when(i == 0)
        def _init():
            psum[...] = jnp.zeros_like(psum)
            cnt[...] = jnp.zeros_like(cnt)

        dv = dinv_ref[...]
        h2 = jnp.maximum(
            dv * (a2_ref[0] + a2_ref[1] + y2_ref[...]) + b2_ref[...], 0.0)
        gids = lax.broadcasted_iota(jnp.int32, (1, num_graphs), 1)
        onehot = (batch_ref[...] == gids).astype(F32)
        dn = (((0,), (0,)), ((), ()))
        psum[...] += lax.dot_general(onehot, h2, dn,
                                     preferred_element_type=F32)
        cnt[...] += lax.dot_general(onehot, jnp.ones_like(h2), dn,
                                    preferred_element_type=F32)

        @pl.when(i == num_blocks - 1)
        def _final():
            p = psum[...] / jnp.maximum(cnt[...], 1.0)
            z = jnp.maximum(
                jnp.dot(p, wc1_ref[...], preferred_element_type=F32)
                + bc1_ref[...], 0.0)
            out_ref[...] = (jnp.dot(z, wc2_ref[...],
                                    preferred_element_type=F32)
                            + bc2_ref[...])

    return body


def kernel(x, edge_index, batch, W1, b1, W2, b2, Wc1, bc1, Wc2, bc2):
    n, dmod = x.shape
    e = edge_index.shape[1]
    num_graphs = 64
    out_dim = Wc2.shape[1]
    h = W1.shape[1]
    assert dmod == 128 and h == 128

    deg_kernel, agg_kernel, nc, npad, K, RC, erows = _build_sc_kernels(n, e)

    src = edge_index[0]
    dst = edge_index[1]
    epad = erows * K
    srcm = jnp.concatenate(
        [src, jnp.zeros((epad - e,), jnp.int32)]).reshape(erows, K)
    dstm = jnp.concatenate(
        [dst, jnp.full((epad - e,), n, jnp.int32)]).reshape(erows, K)
    zeros_seg = jnp.zeros((2048,), F32)
    ones_k = jnp.ones((K,), F32)
    zrows = jnp.zeros((RC, 128), F32)

    degp = deg_kernel(dstm, zeros_seg, ones_k)
    degt = jnp.transpose(degp.reshape(nc, npad)[:, :n])

    RB = 2000
    nb = n // RB
    grid = (nb,)

    y1, dinv = pl.pallas_call(
        _tc1_body,
        grid=grid,
        in_specs=[
            pl.BlockSpec((RB, 128), lambda i: (i, 0)),
            pl.BlockSpec((128, 128), lambda i: (0, 0)),
            pl.BlockSpec((RB, nc), lambda i: (i, 0)),
        ],
        out_specs=[
            pl.BlockSpec((RB, 128), lambda i: (i, 0)),
            pl.BlockSpec((RB, 1), lambda i: (i, 0)),
        ],
        out_shape=[
            jax.ShapeDtypeStruct((n, 128), F32),
            jax.ShapeDtypeStruct((n, 1), F32),
        ],
    )(x, W1, degt)

    acc1 = agg_kernel(y1, srcm.reshape(-1), dstm.reshape(-1),
                      zrows).reshape(nc, n, 128)

    y2 = pl.pallas_call(
        _tc2_body,
        grid=grid,
        in_specs=[
            pl.BlockSpec((nc, RB, 128), lambda i: (0, i, 0)),
            pl.BlockSpec((RB, 128), lambda i: (i, 0)),
            pl.BlockSpec((RB, 1), lambda i: (i, 0)),
            pl.BlockSpec((128, 128), lambda i: (0, 0)),
            pl.BlockSpec((1, 128), lambda i: (0, 0)),
        ],
        out_specs=pl.BlockSpec((RB, 128), lambda i: (i, 0)),
        out_shape=jax.ShapeDtypeStruct((n, 128), F32),
    )(acc1, y1, dinv, W2, b1.reshape(1, -1))

    acc2 = agg_kernel(y2, srcm.reshape(-1), dstm.reshape(-1),
                      zrows).reshape(nc, n, 128)

    out = pl.pallas_call(
        _make_tc3_body(nb, num_graphs),
        grid=grid,
        in_specs=[
            pl.BlockSpec((nc, RB, 128), lambda i: (0, i, 0)),
            pl.BlockSpec((RB, 128), lambda i: (i, 0)),
            pl.BlockSpec((RB, 1), lambda i: (i, 0)),
            pl.BlockSpec((1, 128), lambda i: (0, 0)),
            pl.BlockSpec((RB, 1), lambda i: (i, 0)),
            pl.BlockSpec((128, Wc1.shape[1]), lambda i: (0, 0)),
            pl.BlockSpec((1, Wc1.shape[1]), lambda i: (0, 0)),
            pl.BlockSpec((Wc2.shape[0], out_dim), lambda i: (0, 0)),
            pl.BlockSpec((1, out_dim), lambda i: (0, 0)),
        ],
        out_specs=pl.BlockSpec((num_graphs, out_dim), lambda i: (0, 0)),
        out_shape=jax.ShapeDtypeStruct((num_graphs, out_dim), F32),
        scratch_shapes=[
            pltpu.VMEM((num_graphs, 128), F32),
            pltpu.VMEM((num_graphs, 128), F32),
        ],
    )(acc2, y2, dinv, b2.reshape(1, -1), batch.reshape(-1, 1),
      Wc1, bc1.reshape(1, -1), Wc2, bc2.reshape(1, -1))

    return out


# speedup vs baseline: 2.0467x; 1.7203x over previous
import functools

import jax
import jax.numpy as jnp
from jax import lax
from jax.experimental import pallas as pl
from jax.experimental.pallas import tpu as pltpu
from jax.experimental.pallas import tpu_sc as plsc

F32 = jnp.float32


def _sc_geometry():
    try:
        info = plsc.get_sparse_core_info()
        return info.num_cores, info.num_subcores
    except Exception:
        return 2, 16


def _build_sc_kernels(n, e, pipelined):
    nc, ns = _sc_geometry()
    nw = nc * ns
    K = 128
    nchunks = e // K
    assert e % K == 0
    full, rem = divmod(nchunks, nw)
    ZC = 2048
    npad = -(-n // ZC) * ZC
    nzseg = npad // ZC
    assert nzseg <= ns
    mesh = plsc.VectorSubcoreMesh(core_axis_name="c", subcore_axis_name="s")

    @functools.partial(
        pl.kernel,
        out_type=jax.ShapeDtypeStruct((nc * npad,), F32),
        mesh=mesh,
        scratch_types=[
            pltpu.VMEM_SHARED((npad,), F32),
            pltpu.VMEM((K,), jnp.int32),
            pltpu.VMEM((K,), F32),
            pltpu.VMEM((ZC,), F32),
        ],
    )
    def deg_kernel(dst_hbm, zeros_hbm, ones_hbm, out_hbm, acc, idxv, onesv,
                   stage):
        c = lax.axis_index("c")
        s = lax.axis_index("s")
        wid = s * nc + c

        @pl.when(s < nzseg)
        def _zero():
            pltpu.sync_copy(zeros_hbm, stage)
            pltpu.sync_copy(stage, acc.at[pl.ds(s * ZC, ZC)])

        pltpu.sync_copy(ones_hbm, onesv)
        plsc.subcore_barrier()

        def body(i, carry):
            base = (i * nw + wid) * K
            pltpu.sync_copy(dst_hbm.at[pl.ds(base, K)], idxv)
            pltpu.sync_copy(onesv, acc.at[idxv], add=True)
            return carry

        niter = full + jnp.where(wid < rem, 1, 0)
        lax.fori_loop(0, niter, body, 0)
        plsc.subcore_barrier()

        @pl.when(s < nzseg)
        def _readout():
            pltpu.sync_copy(acc.at[pl.ds(s * ZC, ZC)], stage)
            pltpu.sync_copy(stage, out_hbm.at[pl.ds(c * npad + s * ZC, ZC)])

    RC = 80
    nrc = n // RC
    assert n % RC == 0
    rfull, rrem = divmod(nrc, ns)

    agg_scratch = [
        pltpu.VMEM_SHARED((n, 128), F32),
        pltpu.VMEM((K,), jnp.int32),
        pltpu.VMEM((K,), jnp.int32),
        pltpu.VMEM((K, 128), F32),
        pltpu.VMEM((RC, 128), F32),
    ]
    if pipelined:
        agg_scratch += [
            pltpu.VMEM((K,), jnp.int32),
            pltpu.VMEM((K,), jnp.int32),
            pltpu.VMEM((K, 128), F32),
            pltpu.SemaphoreType.DMA,
            pltpu.SemaphoreType.DMA,
        ]

    def _zero_phase(zrows_hbm, acc, stage, s, nrchunk):
        pltpu.sync_copy(zrows_hbm, stage)

        def zero_chunk(i, carry):
            r0 = (i * ns + s) * RC
            pltpu.sync_copy(stage, acc.at[pl.ds(r0, RC)])
            return carry

        lax.fori_loop(0, nrchunk, zero_chunk, 0)

    def _read_phase(out_hbm, acc, stage, c, s, nrchunk):
        def read_chunk(i, carry):
            r0 = (i * ns + s) * RC
            pltpu.sync_copy(acc.at[pl.ds(r0, RC)], stage)
            pltpu.sync_copy(stage, out_hbm.at[pl.ds(c * n + r0, RC)])
            return carry

        lax.fori_loop(0, nrchunk, read_chunk, 0)

    if not pipelined:
        @functools.partial(
            pl.kernel,
            out_type=jax.ShapeDtypeStruct((nc * n, 128), F32),
            mesh=mesh,
            scratch_types=agg_scratch,
        )
        def agg_kernel(y_hbm, src_hbm, dst_hbm, zrows_hbm, out_hbm,
                       acc, idxs, idxd, rows, stage):
            c = lax.axis_index("c")
            s = lax.axis_index("s")
            wid = s * nc + c
            nrchunk = rfull + jnp.where(s < rrem, 1, 0)
            _zero_phase(zrows_hbm, acc, stage, s, nrchunk)
            plsc.subcore_barrier()

            def body(i, carry):
                base = (i * nw + wid) * K
                pltpu.sync_copy(src_hbm.at[pl.ds(base, K)], idxs)
                pltpu.sync_copy(dst_hbm.at[pl.ds(base, K)], idxd)
                pltpu.sync_copy(y_hbm.at[idxs], rows)
                pltpu.sync_copy(rows, acc.at[idxd], add=True)
                return carry

            niter = full + jnp.where(wid < rem, 1, 0)
            lax.fori_loop(0, niter, body, 0)
            plsc.subcore_barrier()
            _read_phase(out_hbm, acc, stage, c, s, nrchunk)
    else:
        @functools.partial(
            pl.kernel,
            out_type=jax.ShapeDtypeStruct((nc * n, 128), F32),
            mesh=mesh,
            scratch_types=agg_scratch,
        )
        def agg_kernel(y_hbm, src_hbm, dst_hbm, zrows_hbm, out_hbm,
                       acc, idxs0, idxd0, rows0, stage,
                       idxs1, idxd1, rows1, sem0, sem1):
            c = lax.axis_index("c")
            s = lax.axis_index("s")
            wid = s * nc + c
            nrchunk = rfull + jnp.where(s < rrem, 1, 0)
            niter = full + jnp.where(wid < rem, 1, 0)

            base0 = wid * K
            pltpu.sync_copy(src_hbm.at[pl.ds(base0, K)], idxs0)
            pltpu.sync_copy(dst_hbm.at[pl.ds(base0, K)], idxd0)
            pltpu.async_copy(y_hbm.at[idxs0], rows0, sem0)

            _zero_phase(zrows_hbm, acc, stage, s, nrchunk)
            plsc.subcore_barrier()

            def body(i, carry):
                nxt = i + 1
                go = nxt < niter

                @pl.when(go & (nxt % 2 == 1))
                def _i1():
                    b = (nxt * nw + wid) * K
                    pltpu.sync_copy(src_hbm.at[pl.ds(b, K)], idxs1)
                    pltpu.sync_copy(dst_hbm.at[pl.ds(b, K)], idxd1)
                    pltpu.async_copy(y_hbm.at[idxs1], rows1, sem1)

                @pl.when(go & (nxt % 2 == 0))
                def _i0():
                    b = (nxt * nw + wid) * K
                    pltpu.sync_copy(src_hbm.at[pl.ds(b, K)], idxs0)
                    pltpu.sync_copy(dst_hbm.at[pl.ds(b, K)], idxd0)
                    pltpu.async_copy(y_hbm.at[idxs0], rows0, sem0)

                @pl.when(i % 2 == 0)
                def _s0():
                    pltpu.make_async_copy(y_hbm.at[pl.ds(0, K)], rows0,
                                          sem0).wait()
                    pltpu.sync_copy(rows0, acc.at[idxd0], add=True)

                @pl.when(i % 2 == 1)
                def _s1():
                    pltpu.make_async_copy(y_hbm.at[pl.ds(0, K)], rows1,
                                          sem1).wait()
                    pltpu.sync_copy(rows1, acc.at[idxd1], add=True)

                return carry

            lax.fori_loop(0, niter, body, 0)
            plsc.subcore_barrier()
            _read_phase(out_hbm, acc, stage, c, s, nrchunk)

    return deg_kernel, agg_kernel, nc, npad, K, RC


def _tc1_body(x_ref, w1_ref, degt_ref, y1_ref, dinv_ref):
    d = degt_ref[:, 0:1] + degt_ref[:, 1:2] + 1.0
    dv = lax.rsqrt(d)
    m1 = jnp.dot(x_ref[...], w1_ref[...], preferred_element_type=F32)
    y1_ref[...] = dv * m1
    dinv_ref[...] = dv


def _tc2_body(a1_ref, y1_ref, dinv_ref, w2_ref, b1_ref, y2_ref):
    dv = dinv_ref[...]
    z1 = dv * (a1_ref[0] + a1_ref[1] + y1_ref[...]) + b1_ref[...]
    h1 = jnp.maximum(z1, 0.0)
    m2 = jnp.dot(h1, w2_ref[...], preferred_element_type=F32)
    y2_ref[...] = dv * m2


def _make_tc3_body(num_blocks, num_graphs):
    def body(a2_ref, y2_ref, dinv_ref, b2_ref, batch_ref,
             wc1_ref, bc1_ref, wc2_ref, bc2_ref, out_ref, psum, cnt):
        i = pl.program_id(0)

        @pl.when(i == 0)
        def _init():
            psum[...] = jnp.zeros_like(psum)
            cnt[...] = jnp.zeros_like(cnt)

        dv = dinv_ref[...]
        h2 = jnp.maximum(
            dv * (a2_ref[0] + a2_ref[1] + y2_ref[...]) + b2_ref[...], 0.0)
        gids = lax.broadcasted_iota(jnp.int32, (1, num_graphs), 1)
        onehot = (batch_ref[...] == gids).astype(F32)
        dn = (((0,), (0,)), ((), ()))
        psum[...] += lax.dot_general(onehot, h2, dn,
                                     preferred_element_type=F32)
        cnt[...] += lax.dot_general(onehot, jnp.ones_like(h2), dn,
                                    preferred_element_type=F32)

        @pl.when(i == num_blocks - 1)
        def _final():
            p = psum[...] / jnp.maximum(cnt[...], 1.0)
            z = jnp.maximum(
                jnp.dot(p, wc1_ref[...], preferred_element_type=F32)
                + bc1_ref[...], 0.0)
            out_ref[...] = (jnp.dot(z, wc2_ref[...],
                                    preferred_element_type=F32)
                            + bc2_ref[...])

    return body


_PIPELINED = False


def kernel(x, edge_index, batch, W1, b1, W2, b2, Wc1, bc1, Wc2, bc2):
    n, dmod = x.shape
    e = edge_index.shape[1]
    num_graphs = 64
    out_dim = Wc2.shape[1]
    h = W1.shape[1]
    assert dmod == 128 and h == 128

    deg_kernel, agg_kernel, nc, npad, K, RC = _build_sc_kernels(
        n, e, _PIPELINED)

    src = edge_index[0]
    dst = edge_index[1]
    zeros_seg = jnp.zeros((2048,), F32)
    ones_k = jnp.ones((K,), F32)
    zrows = jnp.zeros((RC, 128), F32)

    degp = deg_kernel(dst, zeros_seg, ones_k)
    degt = jnp.transpose(degp.reshape(nc, npad)[:, :n])

    RB = 2000
    nb = n // RB
    grid = (nb,)

    y1, dinv = pl.pallas_call(
        _tc1_body,
        grid=grid,
        in_specs=[
            pl.BlockSpec((RB, 128), lambda i: (i, 0)),
            pl.BlockSpec((128, 128), lambda i: (0, 0)),
            pl.BlockSpec((RB, nc), lambda i: (i, 0)),
        ],
        out_specs=[
            pl.BlockSpec((RB, 128), lambda i: (i, 0)),
            pl.BlockSpec((RB, 1), lambda i: (i, 0)),
        ],
        out_shape=[
            jax.ShapeDtypeStruct((n, 128), F32),
            jax.ShapeDtypeStruct((n, 1), F32),
        ],
    )(x, W1, degt)

    acc1 = agg_kernel(y1, src, dst, zrows).reshape(nc, n, 128)

    y2 = pl.pallas_call(
        _tc2_body,
        grid=grid,
        in_specs=[
            pl.BlockSpec((nc, RB, 128), lambda i: (0, i, 0)),
            pl.BlockSpec((RB, 128), lambda i: (i, 0)),
            pl.BlockSpec((RB, 1), lambda i: (i, 0)),
            pl.BlockSpec((128, 128), lambda i: (0, 0)),
            pl.BlockSpec((1, 128), lambda i: (0, 0)),
        ],
        out_specs=pl.BlockSpec((RB, 128), lambda i: (i, 0)),
        out_shape=jax.ShapeDtypeStruct((n, 128), F32),
    )(acc1, y1, dinv, W2, b1.reshape(1, -1))

    acc2 = agg_kernel(y2, src, dst, zrows).reshape(nc, n, 128)

    out = pl.pallas_call(
        _make_tc3_body(nb, num_graphs),
        grid=grid,
        in_specs=[
            pl.BlockSpec((nc, RB, 128), lambda i: (0, i, 0)),
            pl.BlockSpec((RB, 128), lambda i: (i, 0)),
            pl.BlockSpec((RB, 1), lambda i: (i, 0)),
            pl.BlockSpec((1, 128), lambda i: (0, 0)),
            pl.BlockSpec((RB, 1), lambda i: (i, 0)),
            pl.BlockSpec((128, Wc1.shape[1]), lambda i: (0, 0)),
            pl.BlockSpec((1, Wc1.shape[1]), lambda i: (0, 0)),
            pl.BlockSpec((Wc2.shape[0], out_dim), lambda i: (0, 0)),
            pl.BlockSpec((1, out_dim), lambda i: (0, 0)),
        ],
        out_specs=pl.BlockSpec((num_graphs, out_dim), lambda i: (0, 0)),
        out_shape=jax.ShapeDtypeStruct((num_graphs, out_dim), F32),
        scratch_shapes=[
            pltpu.VMEM((num_graphs, 128), F32),
            pltpu.VMEM((num_graphs, 128), F32),
        ],
    )(acc2, y2, dinv, b2.reshape(1, -1), batch.reshape(-1, 1),
      Wc1, bc1.reshape(1, -1), Wc2, bc2.reshape(1, -1))

    return out


# speedup vs baseline: 2.9799x; 1.4559x over previous
import functools

import jax
import jax.numpy as jnp
from jax import lax
from jax.experimental import pallas as pl
from jax.experimental.pallas import tpu as pltpu
from jax.experimental.pallas import tpu_sc as plsc

F32 = jnp.float32


def _sc_geometry():
    try:
        info = plsc.get_sparse_core_info()
        return info.num_cores, info.num_subcores
    except Exception:
        return 2, 16


def _build_sc_kernels(n, e, pipelined):
    nc, ns = _sc_geometry()
    nw = nc * ns
    K = 128
    nchunks = e // K
    assert e % K == 0
    full, rem = divmod(nchunks, nw)
    ZC = 2048
    npad = -(-n // ZC) * ZC
    nzseg = npad // ZC
    assert nzseg <= ns
    mesh = plsc.VectorSubcoreMesh(core_axis_name="c", subcore_axis_name="s")

    @functools.partial(
        pl.kernel,
        out_type=jax.ShapeDtypeStruct((nc * npad,), F32),
        mesh=mesh,
        scratch_types=[
            pltpu.VMEM_SHARED((npad,), F32),
            pltpu.VMEM((K,), jnp.int32),
            pltpu.VMEM((K,), F32),
            pltpu.VMEM((ZC,), F32),
        ],
    )
    def deg_kernel(dst_hbm, zeros_hbm, ones_hbm, out_hbm, acc, idxv, onesv,
                   stage):
        c = lax.axis_index("c")
        s = lax.axis_index("s")
        wid = s * nc + c

        @pl.when(s < nzseg)
        def _zero():
            pltpu.sync_copy(zeros_hbm, stage)
            pltpu.sync_copy(stage, acc.at[pl.ds(s * ZC, ZC)])

        pltpu.sync_copy(ones_hbm, onesv)
        plsc.subcore_barrier()

        def body(i, carry):
            base = (i * nw + wid) * K
            pltpu.sync_copy(dst_hbm.at[pl.ds(base, K)], idxv)
            pltpu.sync_copy(onesv, acc.at[idxv], add=True)
            return carry

        niter = full + jnp.where(wid < rem, 1, 0)
        lax.fori_loop(0, niter, body, 0)
        plsc.subcore_barrier()

        @pl.when(s < nzseg)
        def _readout():
            pltpu.sync_copy(acc.at[pl.ds(s * ZC, ZC)], stage)
            pltpu.sync_copy(stage, out_hbm.at[pl.ds(c * npad + s * ZC, ZC)])

    RC = 80
    nrc = n // RC
    assert n % RC == 0
    rfull, rrem = divmod(nrc, ns)

    agg_scratch = [
        pltpu.VMEM_SHARED((n, 128), F32),
        pltpu.VMEM((K,), jnp.int32),
        pltpu.VMEM((K,), jnp.int32),
        pltpu.VMEM((K, 128), F32),
        pltpu.VMEM((RC, 128), F32),
    ]
    if pipelined:
        agg_scratch += [
            pltpu.VMEM((K,), jnp.int32),
            pltpu.VMEM((K,), jnp.int32),
            pltpu.VMEM((K, 128), F32),
            pltpu.SemaphoreType.DMA,
            pltpu.SemaphoreType.DMA,
        ]

    def _zero_phase(zrows_hbm, acc, stage, s, nrchunk):
        pltpu.sync_copy(zrows_hbm, stage)

        def zero_chunk(i, carry):
            r0 = (i * ns + s) * RC
            pltpu.sync_copy(stage, acc.at[pl.ds(r0, RC)])
            return carry

        lax.fori_loop(0, nrchunk, zero_chunk, 0)

    def _read_phase(out_hbm, acc, stage, c, s, nrchunk):
        def read_chunk(i, carry):
            r0 = (i * ns + s) * RC
            pltpu.sync_copy(acc.at[pl.ds(r0, RC)], stage)
            pltpu.sync_copy(stage, out_hbm.at[pl.ds(c * n + r0, RC)])
            return carry

        lax.fori_loop(0, nrchunk, read_chunk, 0)

    if not pipelined:
        @functools.partial(
            pl.kernel,
            out_type=jax.ShapeDtypeStruct((nc * n, 128), F32),
            mesh=mesh,
            scratch_types=agg_scratch,
        )
        def agg_kernel(y_hbm, src_hbm, dst_hbm, zrows_hbm, out_hbm,
                       acc, idxs, idxd, rows, stage):
            c = lax.axis_index("c")
            s = lax.axis_index("s")
            wid = s * nc + c
            nrchunk = rfull + jnp.where(s < rrem, 1, 0)
            _zero_phase(zrows_hbm, acc, stage, s, nrchunk)
            plsc.subcore_barrier()

            def body(i, carry):
                base = (i * nw + wid) * K
                pltpu.sync_copy(src_hbm.at[pl.ds(base, K)], idxs)
                pltpu.sync_copy(dst_hbm.at[pl.ds(base, K)], idxd)
                pltpu.sync_copy(y_hbm.at[idxs], rows)
                pltpu.sync_copy(rows, acc.at[idxd], add=True)
                return carry

            niter = full + jnp.where(wid < rem, 1, 0)
            lax.fori_loop(0, niter, body, 0)
            plsc.subcore_barrier()
            _read_phase(out_hbm, acc, stage, c, s, nrchunk)
    else:
        @functools.partial(
            pl.kernel,
            out_type=jax.ShapeDtypeStruct((nc * n, 128), F32),
            mesh=mesh,
            scratch_types=agg_scratch,
        )
        def agg_kernel(y_hbm, src_hbm, dst_hbm, zrows_hbm, out_hbm,
                       acc, idxs0, idxd0, rows0, stage,
                       idxs1, idxd1, rows1, sem0, sem1):
            c = lax.axis_index("c")
            s = lax.axis_index("s")
            wid = s * nc + c
            nrchunk = rfull + jnp.where(s < rrem, 1, 0)
            niter = full + jnp.where(wid < rem, 1, 0)

            base0 = wid * K
            pltpu.sync_copy(src_hbm.at[pl.ds(base0, K)], idxs0)
            pltpu.sync_copy(dst_hbm.at[pl.ds(base0, K)], idxd0)
            pltpu.async_copy(y_hbm.at[idxs0], rows0, sem0)

            _zero_phase(zrows_hbm, acc, stage, s, nrchunk)
            plsc.subcore_barrier()

            def body(i, carry):
                nxt = i + 1
                go = nxt < niter

                @pl.when(go & (nxt % 2 == 1))
                def _i1():
                    b = (nxt * nw + wid) * K
                    pltpu.sync_copy(src_hbm.at[pl.ds(b, K)], idxs1)
                    pltpu.sync_copy(dst_hbm.at[pl.ds(b, K)], idxd1)
                    pltpu.async_copy(y_hbm.at[idxs1], rows1, sem1)

                @pl.when(go & (nxt % 2 == 0))
                def _i0():
                    b = (nxt * nw + wid) * K
                    pltpu.sync_copy(src_hbm.at[pl.ds(b, K)], idxs0)
                    pltpu.sync_copy(dst_hbm.at[pl.ds(b, K)], idxd0)
                    pltpu.async_copy(y_hbm.at[idxs0], rows0, sem0)

                @pl.when(i % 2 == 0)
                def _s0():
                    pltpu.make_async_copy(y_hbm.at[pl.ds(0, K)], rows0,
                                          sem0).wait()
                    pltpu.sync_copy(rows0, acc.at[idxd0], add=True)

                @pl.when(i % 2 == 1)
                def _s1():
                    pltpu.make_async_copy(y_hbm.at[pl.ds(0, K)], rows1,
                                          sem1).wait()
                    pltpu.sync_copy(rows1, acc.at[idxd1], add=True)

                return carry

            lax.fori_loop(0, niter, body, 0)
            plsc.subcore_barrier()
            _read_phase(out_hbm, acc, stage, c, s, nrchunk)

    return deg_kernel, agg_kernel, nc, npad, K, RC


def _tc1_body(x_ref, w1_ref, degt_ref, y1_ref, dinv_ref):
    d = degt_ref[:, 0:1] + degt_ref[:, 1:2] + 1.0
    dv = lax.rsqrt(d)
    m1 = jnp.dot(x_ref[...], w1_ref[...], preferred_element_type=F32)
    y1_ref[...] = dv * m1
    dinv_ref[...] = dv


def _tc2_body(a1_ref, y1_ref, dinv_ref, w2_ref, b1_ref, y2_ref):
    dv = dinv_ref[...]
    z1 = dv * (a1_ref[0] + a1_ref[1] + y1_ref[...]) + b1_ref[...]
    h1 = jnp.maximum(z1, 0.0)
    m2 = jnp.dot(h1, w2_ref[...], preferred_element_type=F32)
    y2_ref[...] = dv * m2


def _make_tc3_body(num_blocks, num_graphs):
    def body(a2_ref, y2_ref, dinv_ref, b2_ref, batch_ref,
             wc1_ref, bc1_ref, wc2_ref, bc2_ref, out_ref, psum, cnt):
        i = pl.program_id(0)

        @pl.when(i == 0)
        def _init():
            psum[...] = jnp.zeros_like(psum)
            cnt[...] = jnp.zeros_like(cnt)

        dv = dinv_ref[...]
        h2 = jnp.maximum(
            dv * (a2_ref[0] + a2_ref[1] + y2_ref[...]) + b2_ref[...], 0.0)
        gids = lax.broadcasted_iota(jnp.int32, (1, num_graphs), 1)
        onehot = (batch_ref[...] == gids).astype(F32)
        dn = (((0,), (0,)), ((), ()))
        psum[...] += lax.dot_general(onehot, h2, dn,
                                     preferred_element_type=F32)
        cnt[...] += lax.dot_general(onehot, jnp.ones_like(h2), dn,
                                    preferred_element_type=F32)

        @pl.when(i == num_blocks - 1)
        def _final():
            p = psum[...] / jnp.maximum(cnt[...], 1.0)
            z = jnp.maximum(
                jnp.dot(p, wc1_ref[...], preferred_element_type=F32)
                + bc1_ref[...], 0.0)
            out_ref[...] = (jnp.dot(z, wc2_ref[...],
                                    preferred_element_type=F32)
                            + bc2_ref[...])

    return body


_PIPELINED = True


def kernel(x, edge_index, batch, W1, b1, W2, b2, Wc1, bc1, Wc2, bc2):
    n, dmod = x.shape
    e = edge_index.shape[1]
    num_graphs = 64
    out_dim = Wc2.shape[1]
    h = W1.shape[1]
    assert dmod == 128 and h == 128

    deg_kernel, agg_kernel, nc, npad, K, RC = _build_sc_kernels(
        n, e, _PIPELINED)

    src = edge_index[0]
    dst = edge_index[1]
    zeros_seg = jnp.zeros((2048,), F32)
    ones_k = jnp.ones((K,), F32)
    zrows = jnp.zeros((RC, 128), F32)

    degp = deg_kernel(dst, zeros_seg, ones_k)
    degt = jnp.transpose(degp.reshape(nc, npad)[:, :n])

    RB = 2000
    nb = n // RB
    grid = (nb,)

    y1, dinv = pl.pallas_call(
        _tc1_body,
        grid=grid,
        in_specs=[
            pl.BlockSpec((RB, 128), lambda i: (i, 0)),
            pl.BlockSpec((128, 128), lambda i: (0, 0)),
            pl.BlockSpec((RB, nc), lambda i: (i, 0)),
        ],
        out_specs=[
            pl.BlockSpec((RB, 128), lambda i: (i, 0)),
            pl.BlockSpec((RB, 1), lambda i: (i, 0)),
        ],
        out_shape=[
            jax.ShapeDtypeStruct((n, 128), F32),
            jax.ShapeDtypeStruct((n, 1), F32),
        ],
    )(x, W1, degt)

    acc1 = agg_kernel(y1, src, dst, zrows).reshape(nc, n, 128)

    y2 = pl.pallas_call(
        _tc2_body,
        grid=grid,
        in_specs=[
            pl.BlockSpec((nc, RB, 128), lambda i: (0, i, 0)),
            pl.BlockSpec((RB, 128), lambda i: (i, 0)),
            pl.BlockSpec((RB, 1), lambda i: (i, 0)),
            pl.BlockSpec((128, 128), lambda i: (0, 0)),
            pl.BlockSpec((1, 128), lambda i: (0, 0)),
        ],
        out_specs=pl.BlockSpec((RB, 128), lambda i: (i, 0)),
        out_shape=jax.ShapeDtypeStruct((n, 128), F32),
    )(acc1, y1, dinv, W2, b1.reshape(1, -1))

    acc2 = agg_kernel(y2, src, dst, zrows).reshape(nc, n, 128)

    out = pl.pallas_call(
        _make_tc3_body(nb, num_graphs),
        grid=grid,
        in_specs=[
            pl.BlockSpec((nc, RB, 128), lambda i: (0, i, 0)),
            pl.BlockSpec((RB, 128), lambda i: (i, 0)),
            pl.BlockSpec((RB, 1), lambda i: (i, 0)),
            pl.BlockSpec((1, 128), lambda i: (0, 0)),
            pl.BlockSpec((RB, 1), lambda i: (i, 0)),
            pl.BlockSpec((128, Wc1.shape[1]), lambda i: (0, 0)),
            pl.BlockSpec((1, Wc1.shape[1]), lambda i: (0, 0)),
            pl.BlockSpec((Wc2.shape[0], out_dim), lambda i: (0, 0)),
            pl.BlockSpec((1, out_dim), lambda i: (0, 0)),
        ],
        out_specs=pl.BlockSpec((num_graphs, out_dim), lambda i: (0, 0)),
        out_shape=jax.ShapeDtypeStruct((num_graphs, out_dim), F32),
        scratch_shapes=[
            pltpu.VMEM((num_graphs, 128), F32),
            pltpu.VMEM((num_graphs, 128), F32),
        ],
    )(acc2, y2, dinv, b2.reshape(1, -1), batch.reshape(-1, 1),
      Wc1, bc1.reshape(1, -1), Wc2, bc2.reshape(1, -1))

    return out
